# baseline (speedup 1.0000x reference)
import functools

import jax
import jax.numpy as jnp
from jax import lax
from jax.experimental import pallas as pl
from jax.experimental.pallas import tpu as pltpu
from jax.experimental.pallas import tpu_sc as plsc
from jax._src.pallas import mpmd as _mpmd

F32 = jnp.float32
I32 = jnp.int32

N = 10000
DEG = 16
E = N * DEG
ND = 64
ED = 64
HD = 128
NL = 4

HXW = 128
EMW = 128

B_N = 2000
B_E = 3200
RPB = B_E // DEG
G_N = N // B_N
G_E = E // B_E

NC, NS = 2, 16
NT = NC * NS
EPT = E // NT
CH = 1000
NCHUNK = EPT // CH
CHS = 200
NCHUNKS = EPT // CHS
RPS = 624
REM_BASE = RPS * NS
REM = N - REM_BASE


def _fs(shape):
    return pl.BlockSpec(shape, lambda *_: (0,) * len(shape))


def _silu(v):
    return v * jax.nn.sigmoid(v)


def _ln(v, g, b):
    mu = jnp.mean(v, axis=-1, keepdims=True)
    var = jnp.mean((v - mu) ** 2, axis=-1, keepdims=True)
    return (v - mu) / jnp.sqrt(var + 1e-5) * g + b


def _embed_body(coords, dem, cap, depot, w0, b0, w1, b1, out):
    d2d = jnp.sqrt(jnp.sum((coords[...] - depot[...]) ** 2, axis=1,
                           keepdims=True))
    nf = jnp.concatenate([dem[...] / cap[0, 0], d2d], axis=1)
    h = _silu(nf @ w0[...] + b0[...]) @ w1[...] + b1[...]
    out[...] = jnp.concatenate(
        [h, coords[...], jnp.zeros((coords.shape[0], HXW - ND - 2), F32)],
        axis=1)


def _node_embed(coords, demands, capacity, p):
    w0 = p[0]["w"]; b0 = p[0]["b"].reshape(1, -1)
    w1 = p[1]["w"]; b1 = p[1]["b"].reshape(1, -1)
    depot = coords[0:1]
    dem = demands.reshape(N, 1)
    cap = jnp.asarray(capacity, F32).reshape(1, 1)
    return pl.pallas_call(
        _embed_body,
        grid=(G_N,),
        in_specs=[
            pl.BlockSpec((B_N, 2), lambda i: (i, 0)),
            pl.BlockSpec((B_N, 1), lambda i: (i, 0)),
            _fs((1, 1)), _fs((1, 2)),
            _fs(w0.shape), _fs(b0.shape), _fs(w1.shape), _fs(b1.shape),
        ],
        out_specs=pl.BlockSpec((B_N, HXW), lambda i: (i, 0)),
        out_shape=jax.ShapeDtypeStruct((N, HXW), F32),
    )(coords, dem, cap, depot, w0, b0, w1, b1)


def _edist_body(xs_rows, hxd, ed_out, pmax_out):
    xs = jnp.repeat(xs_rows[...], DEG, axis=0)
    rel = hxd[:, ND:ND + 2] - xs
    ed = jnp.sqrt(jnp.sum(rel ** 2, axis=1, keepdims=True))
    ed_out[...] = ed
    pmax_out[...] = jnp.max(ed).reshape(1, 1, 1)


def _edge_dist(coords, hxd0):
    return pl.pallas_call(
        _edist_body,
        grid=(G_E,),
        in_specs=[
            pl.BlockSpec((RPB, 2), lambda i: (i, 0)),
            pl.BlockSpec((B_E, HXW), lambda i: (i, 0)),
        ],
        out_specs=[
            pl.BlockSpec((B_E, 1), lambda i: (i, 0)),
            pl.BlockSpec((1, 1, 1), lambda i: (i, 0, 0)),
        ],
        out_shape=[
            jax.ShapeDtypeStruct((E, 1), F32),
            jax.ShapeDtypeStruct((G_E, 1, 1), F32),
        ],
    )(coords, hxd0)


def _eembed_body(ed, pmax, w0, b0, w1, b1, out):
    md = jnp.max(pmax[...]) + 1e-8
    ef = jnp.concatenate([ed[...], 1.0 - ed[...] / md], axis=1)
    out[...] = _silu(ef @ w0[...] + b0[...]) @ w1[...] + b1[...]


def _edge_embed(ed, pmax, p):
    w0 = p[0]["w"]; b0 = p[0]["b"].reshape(1, -1)
    w1 = p[1]["w"]; b1 = p[1]["b"].reshape(1, -1)
    return pl.pallas_call(
        _eembed_body,
        grid=(G_E,),
        in_specs=[
            pl.BlockSpec((B_E, 1), lambda i: (i, 0)),
            _fs((G_E, 1, 1)),
            _fs(w0.shape), _fs(b0.shape), _fs(w1.shape), _fs(b1.shape),
        ],
        out_specs=pl.BlockSpec((B_E, ED), lambda i: (i, 0)),
        out_shape=jax.ShapeDtypeStruct((E, ED), F32),
    )(ed, pmax, w0, b0, w1, b1)


def _edge_msg_body(has_eskip, last, hx_rows, hxd, e,
                   w1a, w1b, w1c, w1d, b1, w2, b2,
                   wc1, bc1, wc2, we1a, we1b, be1, we2, be2, eg, eb,
                   *rest):
    if has_eskip:
        e0 = rest[0]
        rest = rest[1:]
    if last:
        (en_out,) = rest
    else:
        em_out, wx_out, wy_out, en_out = rest
    hs = hx_rows[:, 0:ND]
    xs = jnp.repeat(hx_rows[:, ND:ND + 2], DEG, axis=0)
    hd = hxd[:, 0:ND]
    xd = hxd[:, ND:ND + 2]
    rel = xd - xs
    dist = jnp.sqrt(jnp.sum(rel ** 2, axis=1, keepdims=True))
    pre = jnp.repeat(hs @ w1a[...], DEG, axis=0)
    mi = pre + hd @ w1b[...] + dist @ w1c[...] + e[...] @ w1d[...] + b1[...]
    m = _silu(mi)
    m = _silu(m @ w2[...] + b2[...])
    if not last:
        cw = jnp.tanh(_silu(m @ wc1[...] + bc1[...]) @ wc2[...])
        wrd = cw * rel / (dist + 1e-8)
        em_out[...] = m
        wx_out[...] = wrd[:, 0:1]
        wy_out[...] = wrd[:, 1:2]
    ee = _silu(e[...] @ we1a[...] + m @ we1b[...] + be1[...])
    en = _ln(e[...] + ee @ we2[...] + be2[...], eg[...], eb[...])
    if has_eskip:
        en = en + e0[...]
    en_out[...] = en


def _edge_msg(hx, hxd, e, lp, e_skip, last=False):
    wm1 = lp["msg"][0]["w"]
    w1a, w1b, w1c, w1d = wm1[0:ND], wm1[ND:2 * ND], wm1[2 * ND:2 * ND + 1], \
        wm1[2 * ND + 1:]
    b1 = lp["msg"][0]["b"].reshape(1, -1)
    w2 = lp["msg"][1]["w"]; b2 = lp["msg"][1]["b"].reshape(1, -1)
    wc1 = lp["coord"][0]["w"]; bc1 = lp["coord"][0]["b"].reshape(1, -1)
    wc2 = lp["coord"][1]["w"]
    we1 = lp["edge"][0]["w"]
    we1a, we1b = we1[0:ED], we1[ED:]
    be1 = lp["edge"][0]["b"].reshape(1, -1)
    we2 = lp["edge"][1]["w"]; be2 = lp["edge"][1]["b"].reshape(1, -1)
    eg = lp["edge_norm"]["g"].reshape(1, -1)
    eb = lp["edge_norm"]["b"].reshape(1, -1)
    ws = [w1a, w1b, w1c, w1d, b1, w2, b2, wc1, bc1, wc2,
          we1a, we1b, be1, we2, be2, eg, eb]
    in_specs = [
        pl.BlockSpec((RPB, HXW), lambda i: (i, 0)),
        pl.BlockSpec((B_E, HXW), lambda i: (i, 0)),
        pl.BlockSpec((B_E, ED), lambda i: (i, 0)),
    ] + [_fs(w.shape) for w in ws]
    args = [hx, hxd, e] + ws
    if e_skip is not None:
        in_specs.append(pl.BlockSpec((B_E, ED), lambda i: (i, 0)))
        args.append(e_skip)
    if last:
        out_specs = [pl.BlockSpec((B_E, ED), lambda i: (i, 0))]
        out_shape = [jax.ShapeDtypeStruct((E, ED), F32)]
    else:
        out_specs = [
            pl.BlockSpec((B_E, EMW), lambda i: (i, 0)),
            pl.BlockSpec((B_E, 1), lambda i: (i, 0)),
            pl.BlockSpec((B_E, 1), lambda i: (i, 0)),
            pl.BlockSpec((B_E, ED), lambda i: (i, 0)),
        ]
        out_shape = [
            jax.ShapeDtypeStruct((E, EMW), F32),
            jax.ShapeDtypeStruct((E, 1), F32),
            jax.ShapeDtypeStruct((E, 1), F32),
            jax.ShapeDtypeStruct((E, ED), F32),
        ]
    return pl.pallas_call(
        functools.partial(_edge_msg_body, e_skip is not None, last),
        grid=(G_E,),
        in_specs=in_specs,
        out_specs=out_specs,
        out_shape=out_shape,
    )(*args)


def _node_upd_body(has_skip, hx, a0, a1, wx0, wx1, wy0, wy1, c0, c1,
                   wn1a, wn1b, bn1, wn2, bn2, ng, nb, *rest):
    if has_skip:
        wska, wskb, bsk, hx0, out = rest
    else:
        (out,) = rest
    agg = a0[...] + a1[...]
    h = hx[:, 0:ND]
    x = hx[:, ND:ND + 2]
    t = _silu(h @ wn1a[...] + agg @ wn1b[...] + bn1[...])
    hn = _ln(h + t @ wn2[...] + bn2[...], ng[...], nb[...])
    if has_skip:
        hn = hn @ wska[...] + hx0[:, 0:ND] @ wskb[...] + bsk[...]
    cd = jnp.concatenate([wx0[...] + wx1[...], wy0[...] + wy1[...]], axis=1)
    cnt = jnp.clip(c0[...] + c1[...], 1.0, None)
    xn = x + cd / cnt
    out[...] = jnp.concatenate(
        [hn, xn, jnp.zeros((B_N, HXW - ND - 2), F32)], axis=1)


def _node_upd(hx, aggp, lp, skip_p, hx0):
    am, awx, awy, acnt = aggp
    wn1 = lp["node"][0]["w"]
    wn1a, wn1b = wn1[0:ND], wn1[ND:]
    bn1 = lp["node"][0]["b"].reshape(1, -1)
    wn2 = lp["node"][1]["w"]; bn2 = lp["node"][1]["b"].reshape(1, -1)
    ng = lp["node_norm"]["g"].reshape(1, -1)
    nb = lp["node_norm"]["b"].reshape(1, -1)
    ws = [wn1a, wn1b, bn1, wn2, bn2, ng, nb]
    nb1 = lambda: pl.BlockSpec((B_N, 1), lambda i: (i, 0))
    in_specs = [
        pl.BlockSpec((B_N, HXW), lambda i: (i, 0)),
        pl.BlockSpec((B_N, EMW), lambda i: (i, 0)),
        pl.BlockSpec((B_N, EMW), lambda i: (i, 0)),
        nb1(), nb1(), nb1(), nb1(), nb1(), nb1(),
    ] + [_fs(w.shape) for w in ws]
    args = [hx, am[0], am[1],
            awx[0:N].reshape(N, 1), awx[N:].reshape(N, 1),
            awy[0:N].reshape(N, 1), awy[N:].reshape(N, 1),
            acnt[0:N].reshape(N, 1), acnt[N:].reshape(N, 1)] + ws
    if skip_p is not None:
        wsk = skip_p["w"]
        wska, wskb = wsk[0:ND], wsk[ND:]
        bsk = skip_p["b"].reshape(1, -1)
        in_specs += [_fs(wska.shape), _fs(wskb.shape), _fs(bsk.shape),
                     pl.BlockSpec((B_N, HXW), lambda i: (i, 0))]
        args += [wska, wskb, bsk, hx0]
    return pl.pallas_call(
        functools.partial(_node_upd_body, skip_p is not None),
        grid=(G_N,),
        in_specs=in_specs,
        out_specs=pl.BlockSpec((B_N, HXW), lambda i: (i, 0)),
        out_shape=jax.ShapeDtypeStruct((N, HXW), F32),
    )(*args)


def _head_body(e, dst2d, wh1, bh1, wh2, bh2, wh3, bh3, p_out, fi_out,
               heat_out):
    heat_out[...] = jnp.zeros((RPB, N), F32)
    z = _silu(e[...] @ wh1[...] + bh1[...])
    z = _silu(z @ wh2[...] + bh2[...])
    lg = (z @ wh3[...] + bh3[...]).reshape(RPB, DEG)
    d = dst2d[...]
    eq = d[:, :, None] == d[:, None, :]
    kio = lax.broadcasted_iota(I32, (RPB, DEG, DEG), 2)
    last = jnp.max(jnp.where(eq, kio, -1), axis=2)
    l_eff = jnp.sum(
        jnp.where(kio == last[:, :, None], lg[:, None, :], 0.0), axis=2)
    jio = lax.broadcasted_iota(I32, (RPB, DEG), 1)
    kept = last == jio
    mx = jnp.max(l_eff, axis=1, keepdims=True)
    ex = jnp.exp(l_eff - mx)
    zden = jnp.sum(jnp.where(kept, ex, 0.0), axis=1, keepdims=True)
    p_out[...] = ex / zden
    rbase = pl.program_id(0) * RPB
    rows = rbase + lax.broadcasted_iota(I32, (RPB, DEG), 0)
    fi_out[...] = rows * N + d


def _head_softmax(e, dst2d, p):
    wh1 = p[0]["w"]; bh1 = p[0]["b"].reshape(1, -1)
    wh2 = p[1]["w"]; bh2 = p[1]["b"].reshape(1, -1)
    wh3 = p[2]["w"]; bh3 = p[2]["b"].reshape(1, -1)
    return pl.pallas_call(
        _head_body,
        grid=(G_E,),
        in_specs=[
            pl.BlockSpec((B_E, ED), lambda i: (i, 0)),
            pl.BlockSpec((RPB, DEG), lambda i: (i, 0)),
            _fs(wh1.shape), _fs(bh1.shape), _fs(wh2.shape), _fs(bh2.shape),
            _fs(wh3.shape), _fs(bh3.shape),
        ],
        out_specs=[
            pl.BlockSpec((RPB, DEG), lambda i: (i, 0)),
            pl.BlockSpec((RPB, DEG), lambda i: (i, 0)),
            pl.BlockSpec((RPB, N), lambda i: (i, 0)),
        ],
        out_shape=[
            jax.ShapeDtypeStruct((N, DEG), F32),
            jax.ShapeDtypeStruct((N, DEG), I32),
            jax.ShapeDtypeStruct((N, N), F32),
        ],
    )(e, dst2d, wh1, bh1, wh2, bh2, wh3, bh3)


def _zero_body(out):
    out[...] = jnp.zeros_like(out)


def _zero_fill():
    return pl.pallas_call(
        _zero_body,
        grid=(125,),
        out_specs=pl.BlockSpec((80, N), lambda i: (i, 0)),
        out_shape=jax.ShapeDtypeStruct((N, N), F32),
    )()


@functools.cache
def _sc_mesh():
    return plsc.VectorSubcoreMesh(
        core_axis_name="c", subcore_axis_name="s",
        num_cores=NC, num_subcores=NS)


def _sc_gather_body(hx_hbm, dst_hbm, out_hbm, idx_v, rows_v, sem):
    c = lax.axis_index("c")
    s = lax.axis_index("s")
    tbase = (c * NS + s) * EPT

    def step(i, _):
        off = tbase + i * CH
        pltpu.sync_copy(dst_hbm.at[pl.ds(off, CH)], idx_v)
        pltpu.async_copy(hx_hbm.at[idx_v], rows_v, sem).wait()
        pltpu.sync_copy(rows_v, out_hbm.at[pl.ds(off, CH)])
        return 0

    lax.fori_loop(0, NCHUNK, step, 0)


@functools.cache
def _sc_gather():
    return pl.kernel(
        _sc_gather_body,
        out_type=jax.ShapeDtypeStruct((E, HXW), F32),
        mesh=_sc_mesh(),
        scratch_types=[
            pltpu.VMEM((CH,), I32),
            pltpu.VMEM((CH, HXW), F32),
            pltpu.SemaphoreType.DMA,
        ],
    )


def _sc_segsum_body(em_hbm, wx_hbm, wy_hbm, on_hbm, dst_hbm,
                    am_out, awx_out, awy_out, acnt_out,
                    em_v, wx_v, wy_v, on_v, idx_v, z1_v, dump_v,
                    sA, sB, sC, sD, sE, sF, sG, sH,
                    acc_m, acc_wx, acc_wy, acc_c):
    c = lax.axis_index("c")
    s = lax.axis_index("s")
    rbase = s * RPS

    def zrow(r, _):
        def zcol(j, _):
            em_v[r, pl.ds(j * 16, 16)] = jnp.zeros((16,), F32)
            return 0
        lax.fori_loop(0, EMW // 16, zcol, 0)
        return 0

    lax.fori_loop(0, CHS, zrow, 0)

    def z1col(j, _):
        z1_v[pl.ds(j * 16, 16)] = jnp.zeros((16,), F32)
        return 0

    lax.fori_loop(0, 13, z1col, 0)

    z0 = pltpu.async_copy(em_v.at[pl.ds(0, 156)],
                          acc_m.at[pl.ds(rbase, 156)], sA)
    z1 = pltpu.async_copy(em_v.at[pl.ds(0, 156)],
                          acc_m.at[pl.ds(rbase + 156, 156)], sB)
    z2 = pltpu.async_copy(em_v.at[pl.ds(0, 156)],
                          acc_m.at[pl.ds(rbase + 312, 156)], sC)
    z3 = pltpu.async_copy(em_v.at[pl.ds(0, 156)],
                          acc_m.at[pl.ds(rbase + 468, 156)], sD)
    za = pltpu.async_copy(z1_v, acc_wx.at[pl.ds(rbase, 208)], sE)
    zb = pltpu.async_copy(z1_v, acc_wy.at[pl.ds(rbase, 208)], sF)
    zc = pltpu.async_copy(z1_v, acc_c.at[pl.ds(rbase, 208)], sG)
    for h in (z0, z1, z2, z3, za, zb, zc):
        h.wait()
    for accv, sm in ((acc_wx, sE), (acc_wy, sF), (acc_c, sG)):
        h1 = pltpu.async_copy(z1_v, accv.at[pl.ds(rbase + 208, 208)], sm)
        h2 = pltpu.async_copy(z1_v, accv.at[pl.ds(rbase + 416, 208)], sA)
        h1.wait(); h2.wait()

    @pl.when(s == 0)
    def _():
        pltpu.sync_copy(em_v.at[pl.ds(0, REM)], acc_m.at[pl.ds(REM_BASE, REM)])
        for accv in (acc_wx, acc_wy, acc_c):
            pltpu.sync_copy(z1_v.at[pl.ds(0, REM)],
                            accv.at[pl.ds(REM_BASE, REM)])

    tbase = (c * NS + s) * EPT
    pltpu.sync_copy(on_hbm.at[pl.ds(0, CHS)], on_v)
    plsc.subcore_barrier()

    def step(i, _):
        off = tbase + i * CHS
        li = pltpu.async_copy(dst_hbm.at[pl.ds(off, CHS)], idx_v, sA)
        lm = pltpu.async_copy(em_hbm.at[pl.ds(off, CHS)], em_v, sB)
        lx = pltpu.async_copy(wx_hbm.at[pl.ds(off, CHS)], wx_v, sC)
        ly = pltpu.async_copy(wy_hbm.at[pl.ds(off, CHS)], wy_v, sD)
        li.wait()
        sc_ = pltpu.async_copy(on_v, acc_c.at[idx_v], sH, add=True)
        lm.wait()
        sm = pltpu.async_copy(em_v, acc_m.at[idx_v], sE, add=True)
        lx.wait()
        sx = pltpu.async_copy(wx_v, acc_wx.at[idx_v], sF, add=True)
        ly.wait()
        sy = pltpu.async_copy(wy_v, acc_wy.at[idx_v], sG, add=True)
        sc_.wait(); sm.wait(); sx.wait(); sy.wait()
        return 0

    lax.fori_loop(0, NCHUNKS, step, 0)
    plsc.subcore_barrier()
    fbase = c * N + rbase
    dm = pltpu.async_copy(acc_m.at[pl.ds(rbase, RPS)],
                          am_out.at[c, pl.ds(rbase, RPS)], sA)
    for accv, outv in ((acc_wx, awx_out), (acc_wy, awy_out),
                       (acc_c, acnt_out)):
        pltpu.sync_copy(accv.at[pl.ds(rbase, RPS)], dump_v)
        pltpu.sync_copy(dump_v, outv.at[pl.ds(fbase, RPS)])
    dm.wait()

    @pl.when(s == 0)
    def _():
        frem = c * N + REM_BASE
        pltpu.sync_copy(acc_m.at[pl.ds(REM_BASE, REM)],
                        am_out.at[c, pl.ds(REM_BASE, REM)])
        for accv, outv in ((acc_wx, awx_out), (acc_wy, awy_out),
                           (acc_c, acnt_out)):
            pltpu.sync_copy(accv.at[pl.ds(REM_BASE, REM)],
                            dump_v.at[pl.ds(0, REM)])
            pltpu.sync_copy(dump_v.at[pl.ds(0, REM)],
                            outv.at[pl.ds(frem, REM)])


@functools.cache
def _sc_segsum():
    return pl.kernel(
        _sc_segsum_body,
        out_type=[
            jax.ShapeDtypeStruct((NC, N, EMW), F32),
            jax.ShapeDtypeStruct((NC * N,), F32),
            jax.ShapeDtypeStruct((NC * N,), F32),
            jax.ShapeDtypeStruct((NC * N,), F32),
        ],
        mesh=_sc_mesh(),
        scratch_types=[
            pltpu.VMEM((CHS, EMW), F32),
            pltpu.VMEM((CHS,), F32),
            pltpu.VMEM((CHS,), F32),
            pltpu.VMEM((CHS,), F32),
            pltpu.VMEM((CHS,), I32),
            pltpu.VMEM((208,), F32),
            pltpu.VMEM((RPS,), F32),
            pltpu.SemaphoreType.DMA,
            pltpu.SemaphoreType.DMA,
            pltpu.SemaphoreType.DMA,
            pltpu.SemaphoreType.DMA,
            pltpu.SemaphoreType.DMA,
            pltpu.SemaphoreType.DMA,
            pltpu.SemaphoreType.DMA,
            pltpu.SemaphoreType.DMA,
            pltpu.VMEM_SHARED((N, EMW), F32),
            pltpu.VMEM_SHARED((N,), F32),
            pltpu.VMEM_SHARED((N,), F32),
            pltpu.VMEM_SHARED((N,), F32),
        ],
    )


def _sc_scatter_body(heat_in, pf_hbm, fi_hbm, heat_out, pv, iv):
    del heat_in
    c = lax.axis_index("c")
    s = lax.axis_index("s")
    tbase = (c * NS + s) * EPT

    def step(i, _):
        off = tbase + i * CH
        pltpu.sync_copy(pf_hbm.at[pl.ds(off, CH)], pv)
        pltpu.sync_copy(fi_hbm.at[pl.ds(off, CH)], iv)
        pltpu.sync_copy(pv, heat_out.at[iv])
        return 0

    lax.fori_loop(0, NCHUNK, step, 0)


@functools.cache
def _sc_scatter():
    return _mpmd._mpmd_map(
        [(_sc_mesh(), _sc_scatter_body)],
        jax.ShapeDtypeStruct((N * N,), F32),
        input_output_aliases={0: 0},
        scratch_types=[
            pltpu.VMEM((CH,), F32),
            pltpu.VMEM((CH,), I32),
        ],
    )


def kernel(coords, demands, capacity, edge_index, params):
    dst = edge_index[1].astype(I32)
    dst2d = dst.reshape(N, DEG)
    ones_e = jnp.ones((E,), F32)

    hx = _node_embed(coords, demands, capacity, params["node_embed"])
    hx0 = hx
    hxd = _sc_gather()(hx, dst)
    ed, pmax = _edge_dist(coords, hxd)
    e = _edge_embed(ed, pmax, params["edge_embed"])
    e0 = e

    for i in range(NL - 1):
        lp = params["layers"][i]
        e_skip = e0 if i == 1 else None
        skip_p = params["skip"][0] if i == 1 else None
        em, wx, wy, e = _edge_msg(hx, hxd, e, lp, e_skip)
        aggp = _sc_segsum()(em, wx.reshape(E), wy.reshape(E), ones_e, dst)
        hx = _node_upd(hx, aggp, lp, skip_p, hx0)
        hxd = _sc_gather()(hx, dst)
    (e,) = (_edge_msg(hx, hxd, e, params["layers"][NL - 1], None, last=True),)
    e = e[0]

    p, fi, heat0 = _head_softmax(e, dst2d, params["head"])
    heat = _sc_scatter()(heat0.reshape(N * N), p.reshape(E), fi.reshape(E))
    return heat.reshape(N, N)


# speedup vs baseline: 1.0311x; 1.0311x over previous
import functools

import jax
import jax.numpy as jnp
from jax import lax
from jax.experimental import pallas as pl
from jax.experimental.pallas import tpu as pltpu
from jax.experimental.pallas import tpu_sc as plsc
from jax._src.pallas import mpmd as _mpmd

F32 = jnp.float32
I32 = jnp.int32

N = 10000
DEG = 16
E = N * DEG
ND = 64
ED = 64
HD = 128
NL = 4

HXW = 128
EMW = 128

B_N = 2000
B_E = 3200
RPB = B_E // DEG
G_N = N // B_N
G_E = E // B_E

NC, NS = 2, 16
NT = NC * NS
EPT = E // NT
CH = 1000
NCHUNK = EPT // CH
CHS = 200
NCHUNKS = EPT // CHS
RPS = 624
REM_BASE = RPS * NS
REM = N - REM_BASE


def _fs(shape):
    return pl.BlockSpec(shape, lambda *_: (0,) * len(shape))


def _silu(v):
    return v * jax.nn.sigmoid(v)


def _ln(v, g, b):
    mu = jnp.mean(v, axis=-1, keepdims=True)
    var = jnp.mean((v - mu) ** 2, axis=-1, keepdims=True)
    return (v - mu) / jnp.sqrt(var + 1e-5) * g + b


def _embed_body(coords, dem, cap, depot, w0, b0, w1, b1, out):
    d2d = jnp.sqrt(jnp.sum((coords[...] - depot[...]) ** 2, axis=1,
                           keepdims=True))
    nf = jnp.concatenate([dem[...] / cap[0, 0], d2d], axis=1)
    h = _silu(nf @ w0[...] + b0[...]) @ w1[...] + b1[...]
    out[...] = jnp.concatenate(
        [h, coords[...], jnp.zeros((coords.shape[0], HXW - ND - 2), F32)],
        axis=1)


def _node_embed(coords, demands, capacity, p):
    w0 = p[0]["w"]; b0 = p[0]["b"].reshape(1, -1)
    w1 = p[1]["w"]; b1 = p[1]["b"].reshape(1, -1)
    depot = coords[0:1]
    dem = demands.reshape(N, 1)
    cap = jnp.asarray(capacity, F32).reshape(1, 1)
    return pl.pallas_call(
        _embed_body,
        grid=(G_N,),
        in_specs=[
            pl.BlockSpec((B_N, 2), lambda i: (i, 0)),
            pl.BlockSpec((B_N, 1), lambda i: (i, 0)),
            _fs((1, 1)), _fs((1, 2)),
            _fs(w0.shape), _fs(b0.shape), _fs(w1.shape), _fs(b1.shape),
        ],
        out_specs=pl.BlockSpec((B_N, HXW), lambda i: (i, 0)),
        out_shape=jax.ShapeDtypeStruct((N, HXW), F32),
    )(coords, dem, cap, depot, w0, b0, w1, b1)


def _edist_body(xs_rows, hxd, ed_out, pmax_out):
    xs = jnp.repeat(xs_rows[...], DEG, axis=0)
    rel = hxd[:, ND:ND + 2] - xs
    ed = jnp.sqrt(jnp.sum(rel ** 2, axis=1, keepdims=True))
    ed_out[...] = ed
    pmax_out[...] = jnp.max(ed).reshape(1, 1, 1)


def _edge_dist(coords, hxd0):
    return pl.pallas_call(
        _edist_body,
        grid=(G_E,),
        in_specs=[
            pl.BlockSpec((RPB, 2), lambda i: (i, 0)),
            pl.BlockSpec((B_E, HXW), lambda i: (i, 0)),
        ],
        out_specs=[
            pl.BlockSpec((B_E, 1), lambda i: (i, 0)),
            pl.BlockSpec((1, 1, 1), lambda i: (i, 0, 0)),
        ],
        out_shape=[
            jax.ShapeDtypeStruct((E, 1), F32),
            jax.ShapeDtypeStruct((G_E, 1, 1), F32),
        ],
    )(coords, hxd0)


def _eembed_body(ed, pmax, w0, b0, w1, b1, out):
    md = jnp.max(pmax[...]) + 1e-8
    ef = jnp.concatenate([ed[...], 1.0 - ed[...] / md], axis=1)
    out[...] = _silu(ef @ w0[...] + b0[...]) @ w1[...] + b1[...]


def _edge_embed(ed, pmax, p):
    w0 = p[0]["w"]; b0 = p[0]["b"].reshape(1, -1)
    w1 = p[1]["w"]; b1 = p[1]["b"].reshape(1, -1)
    return pl.pallas_call(
        _eembed_body,
        grid=(G_E,),
        in_specs=[
            pl.BlockSpec((B_E, 1), lambda i: (i, 0)),
            _fs((G_E, 1, 1)),
            _fs(w0.shape), _fs(b0.shape), _fs(w1.shape), _fs(b1.shape),
        ],
        out_specs=pl.BlockSpec((B_E, ED), lambda i: (i, 0)),
        out_shape=jax.ShapeDtypeStruct((E, ED), F32),
    )(ed, pmax, w0, b0, w1, b1)


def _edge_msg_body(has_eskip, zero_heat, hx_rows, hxd, e,
                   w1a, w1b, w1c, w1d, b1, w2, b2,
                   wc1, bc1, wc2, we1a, we1b, be1, we2, be2, eg, eb,
                   *rest):
    if has_eskip:
        e0 = rest[0]
        rest = rest[1:]
    em_out, wx_out, wy_out, en_out = rest[:4]
    if zero_heat:
        rest[4][...] = jnp.zeros((RPB, N), F32)
    hs = hx_rows[:, 0:ND]
    xs = jnp.repeat(hx_rows[:, ND:ND + 2], DEG, axis=0)
    hd = hxd[:, 0:ND]
    xd = hxd[:, ND:ND + 2]
    rel = xd - xs
    dist = jnp.sqrt(jnp.sum(rel ** 2, axis=1, keepdims=True))
    pre = jnp.repeat(hs @ w1a[...], DEG, axis=0)
    mi = pre + hd @ w1b[...] + dist @ w1c[...] + e[...] @ w1d[...] + b1[...]
    m = _silu(mi)
    m = _silu(m @ w2[...] + b2[...])
    cw = jnp.tanh(_silu(m @ wc1[...] + bc1[...]) @ wc2[...])
    wrd = cw * rel / (dist + 1e-8)
    em_out[...] = m
    wx_out[...] = wrd[:, 0:1]
    wy_out[...] = wrd[:, 1:2]
    ee = _silu(e[...] @ we1a[...] + m @ we1b[...] + be1[...])
    en = _ln(e[...] + ee @ we2[...] + be2[...], eg[...], eb[...])
    if has_eskip:
        en = en + e0[...]
    en_out[...] = en


def _edge_msg(hx, hxd, e, lp, e_skip, zero_heat=False):
    wm1 = lp["msg"][0]["w"]
    w1a, w1b, w1c, w1d = wm1[0:ND], wm1[ND:2 * ND], wm1[2 * ND:2 * ND + 1], \
        wm1[2 * ND + 1:]
    b1 = lp["msg"][0]["b"].reshape(1, -1)
    w2 = lp["msg"][1]["w"]; b2 = lp["msg"][1]["b"].reshape(1, -1)
    wc1 = lp["coord"][0]["w"]; bc1 = lp["coord"][0]["b"].reshape(1, -1)
    wc2 = lp["coord"][1]["w"]
    we1 = lp["edge"][0]["w"]
    we1a, we1b = we1[0:ED], we1[ED:]
    be1 = lp["edge"][0]["b"].reshape(1, -1)
    we2 = lp["edge"][1]["w"]; be2 = lp["edge"][1]["b"].reshape(1, -1)
    eg = lp["edge_norm"]["g"].reshape(1, -1)
    eb = lp["edge_norm"]["b"].reshape(1, -1)
    ws = [w1a, w1b, w1c, w1d, b1, w2, b2, wc1, bc1, wc2,
          we1a, we1b, be1, we2, be2, eg, eb]
    in_specs = [
        pl.BlockSpec((RPB, HXW), lambda i: (i, 0)),
        pl.BlockSpec((B_E, HXW), lambda i: (i, 0)),
        pl.BlockSpec((B_E, ED), lambda i: (i, 0)),
    ] + [_fs(w.shape) for w in ws]
    args = [hx, hxd, e] + ws
    if e_skip is not None:
        in_specs.append(pl.BlockSpec((B_E, ED), lambda i: (i, 0)))
        args.append(e_skip)
    out_specs = [
        pl.BlockSpec((B_E, EMW), lambda i: (i, 0)),
        pl.BlockSpec((B_E, 1), lambda i: (i, 0)),
        pl.BlockSpec((B_E, 1), lambda i: (i, 0)),
        pl.BlockSpec((B_E, ED), lambda i: (i, 0)),
    ]
    out_shape = [
        jax.ShapeDtypeStruct((E, EMW), F32),
        jax.ShapeDtypeStruct((E, 1), F32),
        jax.ShapeDtypeStruct((E, 1), F32),
        jax.ShapeDtypeStruct((E, ED), F32),
    ]
    if zero_heat:
        out_specs.append(pl.BlockSpec((RPB, N), lambda i: (i, 0)))
        out_shape.append(jax.ShapeDtypeStruct((N, N), F32))
    return pl.pallas_call(
        functools.partial(_edge_msg_body, e_skip is not None, zero_heat),
        grid=(G_E,),
        in_specs=in_specs,
        out_specs=out_specs,
        out_shape=out_shape,
    )(*args)


def _node_upd_body(has_skip, hx, a0, a1, wx0, wx1, wy0, wy1, c0, c1,
                   wn1a, wn1b, bn1, wn2, bn2, ng, nb, *rest):
    if has_skip:
        wska, wskb, bsk, hx0, out = rest
    else:
        (out,) = rest
    agg = a0[...] + a1[...]
    h = hx[:, 0:ND]
    x = hx[:, ND:ND + 2]
    t = _silu(h @ wn1a[...] + agg @ wn1b[...] + bn1[...])
    hn = _ln(h + t @ wn2[...] + bn2[...], ng[...], nb[...])
    if has_skip:
        hn = hn @ wska[...] + hx0[:, 0:ND] @ wskb[...] + bsk[...]
    cd = jnp.concatenate([wx0[...] + wx1[...], wy0[...] + wy1[...]], axis=1)
    cnt = jnp.clip(c0[...] + c1[...], 1.0, None)
    xn = x + cd / cnt
    out[...] = jnp.concatenate(
        [hn, xn, jnp.zeros((B_N, HXW - ND - 2), F32)], axis=1)


def _node_upd(hx, aggp, lp, skip_p, hx0):
    am, awx, awy, acnt = aggp
    wn1 = lp["node"][0]["w"]
    wn1a, wn1b = wn1[0:ND], wn1[ND:]
    bn1 = lp["node"][0]["b"].reshape(1, -1)
    wn2 = lp["node"][1]["w"]; bn2 = lp["node"][1]["b"].reshape(1, -1)
    ng = lp["node_norm"]["g"].reshape(1, -1)
    nb = lp["node_norm"]["b"].reshape(1, -1)
    ws = [wn1a, wn1b, bn1, wn2, bn2, ng, nb]
    nb1 = lambda: pl.BlockSpec((B_N, 1), lambda i: (i, 0))
    in_specs = [
        pl.BlockSpec((B_N, HXW), lambda i: (i, 0)),
        pl.BlockSpec((B_N, EMW), lambda i: (i, 0)),
        pl.BlockSpec((B_N, EMW), lambda i: (i, 0)),
        nb1(), nb1(), nb1(), nb1(), nb1(), nb1(),
    ] + [_fs(w.shape) for w in ws]
    args = [hx, am[0], am[1],
            awx[0:N].reshape(N, 1), awx[N:].reshape(N, 1),
            awy[0:N].reshape(N, 1), awy[N:].reshape(N, 1),
            acnt[0:N].reshape(N, 1), acnt[N:].reshape(N, 1)] + ws
    if skip_p is not None:
        wsk = skip_p["w"]
        wska, wskb = wsk[0:ND], wsk[ND:]
        bsk = skip_p["b"].reshape(1, -1)
        in_specs += [_fs(wska.shape), _fs(wskb.shape), _fs(bsk.shape),
                     pl.BlockSpec((B_N, HXW), lambda i: (i, 0))]
        args += [wska, wskb, bsk, hx0]
    return pl.pallas_call(
        functools.partial(_node_upd_body, skip_p is not None),
        grid=(G_N,),
        in_specs=in_specs,
        out_specs=pl.BlockSpec((B_N, HXW), lambda i: (i, 0)),
        out_shape=jax.ShapeDtypeStruct((N, HXW), F32),
    )(*args)


def _head_body(e, dst2d, wh1, bh1, wh2, bh2, wh3, bh3, p_out, fi_out):
    z = _silu(e[...] @ wh1[...] + bh1[...])
    z = _silu(z @ wh2[...] + bh2[...])
    lg = (z @ wh3[...] + bh3[...]).reshape(RPB, DEG)
    d = dst2d[...]
    eq = d[:, :, None] == d[:, None, :]
    kio = lax.broadcasted_iota(I32, (RPB, DEG, DEG), 2)
    last = jnp.max(jnp.where(eq, kio, -1), axis=2)
    l_eff = jnp.sum(
        jnp.where(kio == last[:, :, None], lg[:, None, :], 0.0), axis=2)
    jio = lax.broadcasted_iota(I32, (RPB, DEG), 1)
    kept = last == jio
    mx = jnp.max(l_eff, axis=1, keepdims=True)
    ex = jnp.exp(l_eff - mx)
    zden = jnp.sum(jnp.where(kept, ex, 0.0), axis=1, keepdims=True)
    p_out[...] = ex / zden
    rbase = pl.program_id(0) * RPB
    rows = rbase + lax.broadcasted_iota(I32, (RPB, DEG), 0)
    fi_out[...] = rows * N + d


def _head_softmax(e, dst2d, p):
    wh1 = p[0]["w"]; bh1 = p[0]["b"].reshape(1, -1)
    wh2 = p[1]["w"]; bh2 = p[1]["b"].reshape(1, -1)
    wh3 = p[2]["w"]; bh3 = p[2]["b"].reshape(1, -1)
    return pl.pallas_call(
        _head_body,
        grid=(G_E,),
        in_specs=[
            pl.BlockSpec((B_E, ED), lambda i: (i, 0)),
            pl.BlockSpec((RPB, DEG), lambda i: (i, 0)),
            _fs(wh1.shape), _fs(bh1.shape), _fs(wh2.shape), _fs(bh2.shape),
            _fs(wh3.shape), _fs(bh3.shape),
        ],
        out_specs=[
            pl.BlockSpec((RPB, DEG), lambda i: (i, 0)),
            pl.BlockSpec((RPB, DEG), lambda i: (i, 0)),
        ],
        out_shape=[
            jax.ShapeDtypeStruct((N, DEG), F32),
            jax.ShapeDtypeStruct((N, DEG), I32),
        ],
    )(e, dst2d, wh1, bh1, wh2, bh2, wh3, bh3)


def _zero_body(out):
    out[...] = jnp.zeros_like(out)


def _zero_fill():
    return pl.pallas_call(
        _zero_body,
        grid=(125,),
        out_specs=pl.BlockSpec((80, N), lambda i: (i, 0)),
        out_shape=jax.ShapeDtypeStruct((N, N), F32),
    )()


@functools.cache
def _sc_mesh():
    return plsc.VectorSubcoreMesh(
        core_axis_name="c", subcore_axis_name="s",
        num_cores=NC, num_subcores=NS)


def _sc_gather_body(hx_hbm, dst_hbm, out_hbm, idx_v, rows_v, sem):
    c = lax.axis_index("c")
    s = lax.axis_index("s")
    tbase = (c * NS + s) * EPT

    def step(i, _):
        off = tbase + i * CH
        pltpu.sync_copy(dst_hbm.at[pl.ds(off, CH)], idx_v)
        pltpu.async_copy(hx_hbm.at[idx_v], rows_v, sem).wait()
        pltpu.sync_copy(rows_v, out_hbm.at[pl.ds(off, CH)])
        return 0

    lax.fori_loop(0, NCHUNK, step, 0)


@functools.cache
def _sc_gather():
    return pl.kernel(
        _sc_gather_body,
        out_type=jax.ShapeDtypeStruct((E, HXW), F32),
        mesh=_sc_mesh(),
        scratch_types=[
            pltpu.VMEM((CH,), I32),
            pltpu.VMEM((CH, HXW), F32),
            pltpu.SemaphoreType.DMA,
        ],
    )


def _sc_segsum_body(em_hbm, wx_hbm, wy_hbm, on_hbm, dst_hbm,
                    am_out, awx_out, awy_out, acnt_out,
                    em_v, wx_v, wy_v, on_v, idx_v, z1_v, dump_v,
                    sA, sB, sC, sD, sE, sF, sG, sH,
                    acc_m, acc_wx, acc_wy, acc_c):
    c = lax.axis_index("c")
    s = lax.axis_index("s")
    rbase = s * RPS

    def zrow(r, _):
        def zcol(j, _):
            em_v[r, pl.ds(j * 16, 16)] = jnp.zeros((16,), F32)
            return 0
        lax.fori_loop(0, EMW // 16, zcol, 0)
        return 0

    lax.fori_loop(0, CHS, zrow, 0)

    def z1col(j, _):
        z1_v[pl.ds(j * 16, 16)] = jnp.zeros((16,), F32)
        return 0

    lax.fori_loop(0, 13, z1col, 0)

    z0 = pltpu.async_copy(em_v.at[pl.ds(0, 156)],
                          acc_m.at[pl.ds(rbase, 156)], sA)
    z1 = pltpu.async_copy(em_v.at[pl.ds(0, 156)],
                          acc_m.at[pl.ds(rbase + 156, 156)], sB)
    z2 = pltpu.async_copy(em_v.at[pl.ds(0, 156)],
                          acc_m.at[pl.ds(rbase + 312, 156)], sC)
    z3 = pltpu.async_copy(em_v.at[pl.ds(0, 156)],
                          acc_m.at[pl.ds(rbase + 468, 156)], sD)
    za = pltpu.async_copy(z1_v, acc_wx.at[pl.ds(rbase, 208)], sE)
    zb = pltpu.async_copy(z1_v, acc_wy.at[pl.ds(rbase, 208)], sF)
    zc = pltpu.async_copy(z1_v, acc_c.at[pl.ds(rbase, 208)], sG)
    for h in (z0, z1, z2, z3, za, zb, zc):
        h.wait()
    for accv, sm in ((acc_wx, sE), (acc_wy, sF), (acc_c, sG)):
        h1 = pltpu.async_copy(z1_v, accv.at[pl.ds(rbase + 208, 208)], sm)
        h2 = pltpu.async_copy(z1_v, accv.at[pl.ds(rbase + 416, 208)], sA)
        h1.wait(); h2.wait()

    @pl.when(s == 0)
    def _():
        pltpu.sync_copy(em_v.at[pl.ds(0, REM)], acc_m.at[pl.ds(REM_BASE, REM)])
        for accv in (acc_wx, acc_wy, acc_c):
            pltpu.sync_copy(z1_v.at[pl.ds(0, REM)],
                            accv.at[pl.ds(REM_BASE, REM)])

    tbase = (c * NS + s) * EPT
    pltpu.sync_copy(on_hbm.at[pl.ds(0, CHS)], on_v)
    plsc.subcore_barrier()

    def step(i, _):
        off = tbase + i * CHS
        li = pltpu.async_copy(dst_hbm.at[pl.ds(off, CHS)], idx_v, sA)
        lm = pltpu.async_copy(em_hbm.at[pl.ds(off, CHS)], em_v, sB)
        lx = pltpu.async_copy(wx_hbm.at[pl.ds(off, CHS)], wx_v, sC)
        ly = pltpu.async_copy(wy_hbm.at[pl.ds(off, CHS)], wy_v, sD)
        li.wait()
        sc_ = pltpu.async_copy(on_v, acc_c.at[idx_v], sH, add=True)
        lm.wait()
        sm = pltpu.async_copy(em_v, acc_m.at[idx_v], sE, add=True)
        lx.wait()
        sx = pltpu.async_copy(wx_v, acc_wx.at[idx_v], sF, add=True)
        ly.wait()
        sy = pltpu.async_copy(wy_v, acc_wy.at[idx_v], sG, add=True)
        sc_.wait(); sm.wait(); sx.wait(); sy.wait()
        return 0

    lax.fori_loop(0, NCHUNKS, step, 0)
    plsc.subcore_barrier()
    fbase = c * N + rbase
    dm = pltpu.async_copy(acc_m.at[pl.ds(rbase, RPS)],
                          am_out.at[c, pl.ds(rbase, RPS)], sA)
    for accv, outv in ((acc_wx, awx_out), (acc_wy, awy_out),
                       (acc_c, acnt_out)):
        pltpu.sync_copy(accv.at[pl.ds(rbase, RPS)], dump_v)
        pltpu.sync_copy(dump_v, outv.at[pl.ds(fbase, RPS)])
    dm.wait()

    @pl.when(s == 0)
    def _():
        frem = c * N + REM_BASE
        pltpu.sync_copy(acc_m.at[pl.ds(REM_BASE, REM)],
                        am_out.at[c, pl.ds(REM_BASE, REM)])
        for accv, outv in ((acc_wx, awx_out), (acc_wy, awy_out),
                           (acc_c, acnt_out)):
            pltpu.sync_copy(accv.at[pl.ds(REM_BASE, REM)],
                            dump_v.at[pl.ds(0, REM)])
            pltpu.sync_copy(dump_v.at[pl.ds(0, REM)],
                            outv.at[pl.ds(frem, REM)])


@functools.cache
def _sc_segsum():
    return pl.kernel(
        _sc_segsum_body,
        out_type=[
            jax.ShapeDtypeStruct((NC, N, EMW), F32),
            jax.ShapeDtypeStruct((NC * N,), F32),
            jax.ShapeDtypeStruct((NC * N,), F32),
            jax.ShapeDtypeStruct((NC * N,), F32),
        ],
        mesh=_sc_mesh(),
        scratch_types=[
            pltpu.VMEM((CHS, EMW), F32),
            pltpu.VMEM((CHS,), F32),
            pltpu.VMEM((CHS,), F32),
            pltpu.VMEM((CHS,), F32),
            pltpu.VMEM((CHS,), I32),
            pltpu.VMEM((208,), F32),
            pltpu.VMEM((RPS,), F32),
            pltpu.SemaphoreType.DMA,
            pltpu.SemaphoreType.DMA,
            pltpu.SemaphoreType.DMA,
            pltpu.SemaphoreType.DMA,
            pltpu.SemaphoreType.DMA,
            pltpu.SemaphoreType.DMA,
            pltpu.SemaphoreType.DMA,
            pltpu.SemaphoreType.DMA,
            pltpu.VMEM_SHARED((N, EMW), F32),
            pltpu.VMEM_SHARED((N,), F32),
            pltpu.VMEM_SHARED((N,), F32),
            pltpu.VMEM_SHARED((N,), F32),
        ],
    )


def _sc_scatter_body(heat_in, pf_hbm, fi_hbm, heat_out, pv, iv):
    del heat_in
    c = lax.axis_index("c")
    s = lax.axis_index("s")
    tbase = (c * NS + s) * EPT

    def step(i, _):
        off = tbase + i * CH
        pltpu.sync_copy(pf_hbm.at[pl.ds(off, CH)], pv)
        pltpu.sync_copy(fi_hbm.at[pl.ds(off, CH)], iv)
        pltpu.sync_copy(pv, heat_out.at[iv])
        return 0

    lax.fori_loop(0, NCHUNK, step, 0)


@functools.cache
def _sc_scatter():
    return _mpmd._mpmd_map(
        [(_sc_mesh(), _sc_scatter_body)],
        jax.ShapeDtypeStruct((N * N,), F32),
        input_output_aliases={0: 0},
        scratch_types=[
            pltpu.VMEM((CH,), F32),
            pltpu.VMEM((CH,), I32),
        ],
    )


def kernel(coords, demands, capacity, edge_index, params):
    dst = edge_index[1].astype(I32)
    dst2d = dst.reshape(N, DEG)
    ones_e = jnp.ones((E,), F32)

    hx = _node_embed(coords, demands, capacity, params["node_embed"])
    hx0 = hx
    hxd = _sc_gather()(hx, dst)
    ed, pmax = _edge_dist(coords, hxd)
    e = _edge_embed(ed, pmax, params["edge_embed"])
    e0 = e

    heat0 = None
    for i in range(NL):
        lp = params["layers"][i]
        e_skip = e0 if i == 1 else None
        skip_p = params["skip"][0] if i == 1 else None
        outs = _edge_msg(hx, hxd, e, lp, e_skip, zero_heat=(i == NL - 1))
        em, wx, wy, e = outs[:4]
        if i == NL - 1:
            heat0 = outs[4]
        aggp = _sc_segsum()(em, wx.reshape(E), wy.reshape(E), ones_e, dst)
        hx = _node_upd(hx, aggp, lp, skip_p, hx0)
        if i < NL - 1:
            hxd = _sc_gather()(hx, dst)

    p, fi = _head_softmax(e, dst2d, params["head"])
    heat = _sc_scatter()(heat0.reshape(N * N), p.reshape(E), fi.reshape(E))
    return heat.reshape(N, N)
